# Initial kernel scaffold; baseline (speedup 1.0000x reference)
import jax
import jax.numpy as jnp
from jax.experimental import pallas as pl


def kernel(x, x_sub, pos, pos_sub, batch, batch_sub, W1, b1, gw1, gb1, gms1, W2, b2, gw2, gb2, gms2):
    raise NotImplementedError("write your pallas kernel here")


# speedup vs baseline: 2.6624x; 2.6624x over previous
import functools

import jax
import jax.numpy as jnp
from jax import lax
from jax.experimental import pallas as pl
from jax.experimental.pallas import tpu as pltpu
from jax.experimental.pallas import tpu_sc as plsc

_N = 10000
_NSUB = 2500
_OUT = 256
_EPS = 1e-5

_QBLK = 400
_NBLK = _N // _QBLK
_KPAD = 2560
_PAD_COORD = 1e4

_NW = 32
_NPAD = 10240
_ROWS_PER_W = _NPAD // _NW
_CHUNK = 64
_NCHUNK = _ROWS_PER_W // _CHUNK


def _mlp_body(x_ref, w_ref, b_ref, gw_ref, gb_ref, gms_ref, o_ref):
    y = jnp.dot(x_ref[...], w_ref[...], preferred_element_type=jnp.float32)
    y = y + b_ref[...]
    mean = jnp.mean(y, axis=0, keepdims=True)
    c = y - gms_ref[...] * mean
    var = jnp.mean(c * c, axis=0, keepdims=True)
    z = gw_ref[...] * c / jnp.sqrt(var + _EPS) + gb_ref[...]
    o_ref[...] = jnp.maximum(z, 0.0)


def _top2_body(posq_ref, posk_ref, i1_ref, i2_ref, w1_ref, w2_ref):
    d = None
    for c in range(3):
        q = posq_ref[:, c:c + 1]
        k = posk_ref[c:c + 1, :]
        t = q - k
        d = t * t if d is None else d + t * t
    iota = lax.broadcasted_iota(jnp.int32, (_QBLK, _KPAD), 1)
    big = jnp.int32(2**30)
    m1 = jnp.min(d, axis=1, keepdims=True)
    i1 = jnp.min(jnp.where(d == m1, iota, big), axis=1, keepdims=True)
    dm = jnp.where(iota == i1, jnp.float32(jnp.inf), d)
    m2 = jnp.min(dm, axis=1, keepdims=True)
    i2 = jnp.min(jnp.where(dm == m2, iota, big), axis=1, keepdims=True)
    i1_ref[...] = i1
    i2_ref[...] = i2
    w1_ref[...] = 1.0 / jnp.maximum(m1, 1e-16)
    w2_ref[...] = 1.0 / jnp.maximum(m2, 1e-16)


@functools.lru_cache(maxsize=1)
def _make_sc_gather():
    @functools.partial(
        pl.kernel,
        mesh=plsc.VectorSubcoreMesh(core_axis_name="c", subcore_axis_name="s"),
        out_type=[
            jax.ShapeDtypeStruct((_NPAD, _OUT), jnp.float32),
            jax.ShapeDtypeStruct((_NPAD, _OUT), jnp.float32),
        ],
        scratch_types=[
            pltpu.VMEM((_CHUNK,), jnp.int32),
            pltpu.VMEM((_CHUNK,), jnp.int32),
            pltpu.VMEM((_CHUNK, _OUT), jnp.float32),
            pltpu.VMEM((_CHUNK, _OUT), jnp.float32),
            pltpu.SemaphoreType.DMA,
            pltpu.SemaphoreType.DMA,
        ],
    )
    def _sc_gather(table_hbm, idx1_hbm, idx2_hbm, r1_hbm, r2_hbm,
                   i1_v, i2_v, rows1_v, rows2_v, sem1, sem2):
        wid = lax.axis_index("s") * 2 + lax.axis_index("c")
        base = wid * _ROWS_PER_W
        for i in range(_NCHUNK):
            off = base + i * _CHUNK
            pltpu.sync_copy(idx1_hbm.at[pl.ds(off, _CHUNK)], i1_v)
            pltpu.sync_copy(idx2_hbm.at[pl.ds(off, _CHUNK)], i2_v)
            cp1 = pltpu.async_copy(table_hbm.at[i1_v], rows1_v, sem1)
            cp2 = pltpu.async_copy(table_hbm.at[i2_v], rows2_v, sem2)
            cp1.wait()
            cp2.wait()
            pltpu.sync_copy(rows1_v, r1_hbm.at[pl.ds(off, _CHUNK)])
            pltpu.sync_copy(rows2_v, r2_hbm.at[pl.ds(off, _CHUNK)])

    return _sc_gather


def _gather_rows(table, idx1, idx2):
    return _make_sc_gather()(table, idx1, idx2)


def _mlp2_mm_body(x_ref, w_ref, b_ref, y_ref, s_ref):
    y = jnp.dot(x_ref[...], w_ref[...], preferred_element_type=jnp.float32)
    y = y + b_ref[...]
    y_ref[...] = y

    @pl.when(pl.program_id(0) == 0)
    def _init():
        s_ref[...] = jnp.zeros_like(s_ref)

    s_ref[0:1, :] += jnp.sum(y, axis=0, keepdims=True)
    s_ref[1:2, :] += jnp.sum(y * y, axis=0, keepdims=True)


def _norm_blend_body(y_ref, s_ref, gw_ref, gb_ref, gms_ref,
                     r1_ref, r2_ref, w1_ref, w2_ref, o_ref):
    inv_n = jnp.float32(1.0 / _N)
    mean = s_ref[0:1, :] * inv_n
    ey2 = s_ref[1:2, :] * inv_n
    ms = gms_ref[...]
    var = ey2 - ms * (2.0 - ms) * mean * mean
    c = y_ref[...] - ms * mean
    z = gw_ref[...] * c / jnp.sqrt(var + _EPS) + gb_ref[...]
    z = jnp.maximum(z, 0.0)
    w1 = w1_ref[...]
    w2 = w2_ref[...]
    interp = (w1 * r1_ref[...] + w2 * r2_ref[...]) / (w1 + w2)
    o_ref[...] = z + interp


def kernel(x, x_sub, pos, pos_sub, batch, batch_sub,
           W1, b1, gw1, gb1, gms1, W2, b2, gw2, gb2, gms2):
    f32 = jnp.float32

    xs_t = pl.pallas_call(
        _mlp_body,
        out_shape=jax.ShapeDtypeStruct((_NSUB, _OUT), f32),
    )(x_sub, W1, b1.reshape(1, -1), gw1.reshape(1, -1),
      gb1.reshape(1, -1), gms1.reshape(1, -1))

    posk = jnp.full((8, _KPAD), _PAD_COORD, f32)
    posk = posk.at[:3, :_NSUB].set(pos_sub.T)
    i1, i2, w1, w2 = pl.pallas_call(
        _top2_body,
        grid=(_NBLK,),
        in_specs=[
            pl.BlockSpec((_QBLK, 3), lambda i: (i, 0)),
            pl.BlockSpec((8, _KPAD), lambda i: (0, 0)),
        ],
        out_specs=[
            pl.BlockSpec((_QBLK, 1), lambda i: (i, 0)),
            pl.BlockSpec((_QBLK, 1), lambda i: (i, 0)),
            pl.BlockSpec((_QBLK, 1), lambda i: (i, 0)),
            pl.BlockSpec((_QBLK, 1), lambda i: (i, 0)),
        ],
        out_shape=[
            jax.ShapeDtypeStruct((_N, 1), jnp.int32),
            jax.ShapeDtypeStruct((_N, 1), jnp.int32),
            jax.ShapeDtypeStruct((_N, 1), f32),
            jax.ShapeDtypeStruct((_N, 1), f32),
        ],
    )(pos, posk)

    idx1 = jnp.pad(i1.reshape(-1), (0, _NPAD - _N))
    idx2 = jnp.pad(i2.reshape(-1), (0, _NPAD - _N))
    r1, r2 = _gather_rows(xs_t, idx1, idx2)

    y, sums = pl.pallas_call(
        _mlp2_mm_body,
        grid=(_NBLK,),
        in_specs=[
            pl.BlockSpec((_QBLK, _OUT), lambda i: (i, 0)),
            pl.BlockSpec((_OUT, _OUT), lambda i: (0, 0)),
            pl.BlockSpec((1, _OUT), lambda i: (0, 0)),
        ],
        out_specs=[
            pl.BlockSpec((_QBLK, _OUT), lambda i: (i, 0)),
            pl.BlockSpec((8, _OUT), lambda i: (0, 0)),
        ],
        out_shape=[
            jax.ShapeDtypeStruct((_N, _OUT), f32),
            jax.ShapeDtypeStruct((8, _OUT), f32),
        ],
    )(x, W2, b2.reshape(1, -1))

    out = pl.pallas_call(
        _norm_blend_body,
        grid=(_NBLK,),
        in_specs=[
            pl.BlockSpec((_QBLK, _OUT), lambda i: (i, 0)),
            pl.BlockSpec((8, _OUT), lambda i: (0, 0)),
            pl.BlockSpec((1, _OUT), lambda i: (0, 0)),
            pl.BlockSpec((1, _OUT), lambda i: (0, 0)),
            pl.BlockSpec((1, _OUT), lambda i: (0, 0)),
            pl.BlockSpec((_QBLK, _OUT), lambda i: (i, 0)),
            pl.BlockSpec((_QBLK, _OUT), lambda i: (i, 0)),
            pl.BlockSpec((_QBLK, 1), lambda i: (i, 0)),
            pl.BlockSpec((_QBLK, 1), lambda i: (i, 0)),
        ],
        out_specs=pl.BlockSpec((_QBLK, _OUT), lambda i: (i, 0)),
        out_shape=jax.ShapeDtypeStruct((_N, _OUT), f32),
    )(y, sums, gw2.reshape(1, -1), gb2.reshape(1, -1), gms2.reshape(1, -1),
      r1[:_N], r2[:_N], w1, w2)
    return out
